# baseline (speedup 1.0000x reference)
import functools

import jax
import jax.numpy as jnp
from jax import lax
from jax.experimental import pallas as pl
from jax.experimental.pallas import tpu as pltpu
from jax.experimental.pallas import tpu_sc as plsc

EMB_DIM = 64
NC, NS = 2, 16
NW = NC * NS
CHUNK = 128
NBUF = 8
K = 4


@functools.lru_cache(maxsize=None)
def _build_gather(n_chunks: int):
    b_per_w = n_chunks * CHUNK
    n_rows = NW * b_per_w
    mesh = plsc.VectorSubcoreMesh(core_axis_name="c", subcore_axis_name="s")

    def body(idx_hbm, table_hbm, out_hbm, idx_v, *scratch):
        rows = scratch[:NBUF]
        gsems = scratch[NBUF:2 * NBUF]
        wsems = scratch[2 * NBUF:3 * NBUF]
        wid = lax.axis_index("s") * NC + lax.axis_index("c")
        base = wid * b_per_w

        pltpu.sync_copy(idx_hbm.at[wid], idx_v)

        def wait_gather(c, b):
            pltpu.make_async_copy(
                table_hbm.at[idx_v.at[c]], rows[b], gsems[b]).wait()

        def start_write(c, b):
            pltpu.async_copy(
                rows[b], out_hbm.at[pl.ds(base + c * CHUNK, CHUNK)],
                wsems[b])

        def wait_write(c, b):
            pltpu.make_async_copy(
                rows[b], out_hbm.at[pl.ds(base + c * CHUNK, CHUNK)],
                wsems[b]).wait()

        def start_gather(c, b):
            pltpu.async_copy(table_hbm.at[idx_v.at[c]], rows[b], gsems[b])

        for b in range(K):
            start_gather(b, b)

        for c in range(K):
            wait_gather(c, c)
            start_write(c, c)
            start_gather(c + K, c + K)

        @pl.loop(K, n_chunks - K, step=NBUF)
        def _(c0):
            for j in range(NBUF):
                c = c0 + j
                b = (K + j) % NBUF
                pb = (2 * K + j) % NBUF
                wait_gather(c, b)
                start_write(c, b)
                wait_write(c - K, pb)
                start_gather(c + K, pb)

        for cs in range(n_chunks - K, n_chunks):
            b = cs % NBUF
            wait_gather(cs, b)
            start_write(cs, b)

        for cs in range(n_chunks - NBUF, n_chunks):
            wait_write(cs, cs % NBUF)

    return pl.kernel(
        body,
        mesh=mesh,
        compiler_params=pltpu.CompilerParams(use_tc_tiling_on_sc=False),
        out_type=jax.ShapeDtypeStruct((n_rows, EMB_DIM), jnp.float32),
        scratch_types=(
            [pltpu.VMEM((n_chunks, CHUNK), jnp.int32)]
            + [pltpu.VMEM((CHUNK, EMB_DIM), jnp.float32)] * NBUF
            + [pltpu.SemaphoreType.DMA] * (2 * NBUF)
        ),
    )


def kernel(token_ids, weight):
    orig_shape = token_ids.shape
    flat = token_ids.reshape(-1).astype(jnp.int32)
    n = flat.shape[0]
    tile = NW * CHUNK * NBUF
    n_pad = -(-n // tile) * tile
    if n_pad != n:
        flat = jnp.pad(flat, (0, n_pad - n))
    n_chunks = n_pad // (NW * CHUNK)
    idx3 = flat.reshape(NW, n_chunks, CHUNK)
    out = _build_gather(n_chunks)(idx3, weight)
    if n_pad != n:
        out = out[:n]
    return out.reshape(*orig_shape, EMB_DIM)
